# baseline (speedup 1.0000x reference)
import functools

import jax
import jax.numpy as jnp
from jax import lax
from jax.experimental import pallas as pl
from jax.experimental.pallas import tpu as pltpu
from jax.experimental.pallas import tpu_sc as plsc

VOCAB = 100000
D = 128
W = 5
B = 16384

NC = 2
NS = 16
NW = NC * NS
BPW = B // NW
CH = 64
NBUF = BPW // CH
IPB = 128 // CH
NPAIR = NBUF // IPB
LANES = 16
GROUPS = D // LANES


def _mesh():
    return plsc.VectorSubcoreMesh(core_axis_name="c", subcore_axis_name="s")


@functools.partial(
    pl.kernel,
    out_type=jax.ShapeDtypeStruct((B, D), jnp.float32),
    mesh=_mesh(),
    scratch_types=[
        pltpu.VMEM((NPAIR, W, IPB * CH), jnp.int32),
        pltpu.VMEM((NBUF, CH, D), jnp.float32),
        pltpu.VMEM((W, D), jnp.float32),
        pltpu.SemaphoreType.DMA,
        pltpu.SemaphoreType.DMA,
        pltpu.SemaphoreType.DMA,
        pltpu.SemaphoreType.DMA,
        pltpu.SemaphoreType.DMA,
        pltpu.SemaphoreType.DMA,
        pltpu.SemaphoreType.DMA,
        pltpu.SemaphoreType.DMA,
        pltpu.SemaphoreType.DMA,
        pltpu.SemaphoreType.DMA,
    ],
)
def _word_encode(word_hbm, table_hbm, pos_hbm, out_hbm,
                 idx_v, acc_v, pos_v, isem,
                 g0, g1, g2, g3, g4, g5, g6, g7, osem):
    wid = lax.axis_index("s") * NC + lax.axis_index("c")
    base = wid * BPW
    gsem = [g0, g1, g2, g3, g4, g5, g6, g7]

    idx_descs = [
        pltpu.async_copy(word_hbm.at[:, pl.ds(base + p * IPB * CH, IPB * CH)],
                         idx_v.at[p], isem)
        for p in range(NPAIR)
    ]

    pltpu.sync_copy(pos_hbm, pos_v)
    possum = []
    for c in range(GROUPS):
        sl = pl.ds(c * LANES, LANES)
        s = (pos_v[0, sl] + pos_v[1, sl]) + (pos_v[2, sl] + pos_v[3, sl])
        possum.append(s + pos_v[4, sl])

    for dsc in idx_descs:
        dsc.wait()

    def idx_slice(j, r):
        return idx_v.at[j // IPB, r, pl.ds((j % IPB) * CH, CH)]

    descs = {}
    for j in range(NBUF):
        def build(r, ps, j=j):
            for c in range(GROUPS):
                acc_v[j, r, pl.ds(c * LANES, LANES)] = ps[c]
            return ps

        lax.fori_loop(0, CH, build, tuple(possum))
        descs[(j, 0)] = pltpu.async_copy(
            table_hbm.at[idx_slice(j, 0)], acc_v.at[j], gsem[j], add=True)

    for r in range(1, W - 1):
        for j in range(NBUF):
            descs[(j, r - 1)].wait()
            descs[(j, r)] = pltpu.async_copy(
                table_hbm.at[idx_slice(j, r)], acc_v.at[j], gsem[j], add=True)

    def flush(j):
        descs[(j, W - 1)].wait()
        return pltpu.async_copy(
            acc_v.at[j], out_hbm.at[pl.ds(base + j * CH, CH), :], osem)

    outs = []
    for j in range(NBUF):
        descs[(j, W - 2)].wait()
        descs[(j, W - 1)] = pltpu.async_copy(
            table_hbm.at[idx_slice(j, W - 1)], acc_v.at[j], gsem[j], add=True)
        if j >= 2:
            outs.append(flush(j - 2))
    outs.append(flush(NBUF - 2))
    outs.append(flush(NBUF - 1))
    for dsc in outs:
        dsc.wait()


def kernel(word, letter_table, pos_table):
    word = word.astype(jnp.int32)
    return _word_encode(word, letter_table, pos_table)


# speedup vs baseline: 1.0169x; 1.0169x over previous
import functools

import jax
import jax.numpy as jnp
from jax import lax
from jax.experimental import pallas as pl
from jax.experimental.pallas import tpu as pltpu
from jax.experimental.pallas import tpu_sc as plsc

VOCAB = 100000
D = 128
W = 5
B = 16384

NC = 2
NS = 16
NW = NC * NS
BPW = B // NW
CH = 64
NBUF = BPW // CH
IPB = 128 // CH
NPAIR = NBUF // IPB
LANES = 16
GROUPS = D // LANES


def _mesh():
    return plsc.VectorSubcoreMesh(core_axis_name="c", subcore_axis_name="s")


@functools.partial(
    pl.kernel,
    out_type=jax.ShapeDtypeStruct((B, D), jnp.float32),
    mesh=_mesh(),
    scratch_types=[
        pltpu.VMEM((NPAIR, W, IPB * CH), jnp.int32),
        pltpu.VMEM((NBUF, CH, D), jnp.float32),
        pltpu.VMEM((W, D), jnp.float32),
        pltpu.SemaphoreType.DMA,
        pltpu.SemaphoreType.DMA,
        pltpu.SemaphoreType.DMA,
        pltpu.SemaphoreType.DMA,
        pltpu.SemaphoreType.DMA,
        pltpu.SemaphoreType.DMA,
        pltpu.SemaphoreType.DMA,
        pltpu.SemaphoreType.DMA,
        pltpu.SemaphoreType.DMA,
        pltpu.SemaphoreType.DMA,
    ],
)
def _word_encode(word_hbm, table_hbm, pos_hbm, out_hbm,
                 idx_v, acc_v, pos_v, isem,
                 g0, g1, g2, g3, g4, g5, g6, g7, osem):
    wid = lax.axis_index("s") * NC + lax.axis_index("c")
    base = wid * BPW
    gsem = [g0, g1, g2, g3, g4, g5, g6, g7]

    idx_descs = [
        pltpu.async_copy(word_hbm.at[:, pl.ds(base + p * IPB * CH, IPB * CH)],
                         idx_v.at[p], isem)
        for p in range(NPAIR)
    ]

    pltpu.sync_copy(pos_hbm, pos_v)
    possum = []
    for c in range(GROUPS):
        sl = pl.ds(c * LANES, LANES)
        s = (pos_v[0, sl] + pos_v[1, sl]) + (pos_v[2, sl] + pos_v[3, sl])
        possum.append(s + pos_v[4, sl])

    for dsc in idx_descs:
        dsc.wait()

    def idx_slice(j, r):
        return idx_v.at[j // IPB, r, pl.ds((j % IPB) * CH, CH)]

    descs = {}
    for j in range(NBUF):
        def build(r, ps, j=j):
            for c in range(GROUPS):
                acc_v[j, r, pl.ds(c * LANES, LANES)] = ps[c]
            return ps

        lax.fori_loop(0, CH, build, tuple(possum))
        descs[(j, 0)] = pltpu.async_copy(
            table_hbm.at[idx_slice(j, 0)], acc_v.at[j], gsem[j], add=True)

    for r in range(1, W):
        for j in range(NBUF):
            descs[(j, r - 1)].wait()
            descs[(j, r)] = pltpu.async_copy(
                table_hbm.at[idx_slice(j, r)], acc_v.at[j], gsem[j], add=True)

    outs = []
    for j in range(NBUF):
        descs[(j, W - 1)].wait()
        outs.append(pltpu.async_copy(
            acc_v.at[j], out_hbm.at[pl.ds(base + j * CH, CH), :], osem))
    for dsc in outs:
        dsc.wait()


def kernel(word, letter_table, pos_table):
    word = word.astype(jnp.int32)
    return _word_encode(word, letter_table, pos_table)
